# Initial kernel scaffold; baseline (speedup 1.0000x reference)
import jax
import jax.numpy as jnp
from jax.experimental import pallas as pl


def kernel(X, edge_index, W0, b0, W1, b1):
    raise NotImplementedError("write your pallas kernel here")


# speedup vs baseline: 6.9241x; 6.9241x over previous
import functools

import jax
import jax.numpy as jnp
from jax import lax
from jax.experimental import pallas as pl
from jax.experimental.pallas import tpu as pltpu
from jax.experimental.pallas import tpu_sc as plsc

ALPHA = 0.1
N_PROP = 5

NC = 2
NS = 16
LANES = 16

EK = 128
RB = 400


def _mesh():
    return plsc.VectorSubcoreMesh(
        core_axis_name="c", subcore_axis_name="s",
        num_cores=NC, num_subcores=NS)


_SC_PARAMS = pltpu.CompilerParams(use_tc_tiling_on_sc=False)


def _make_deg(N, E):
    nch = E // EK
    tch = (nch + NS - 1) // NS
    nrb = N // RB
    trb = (nrb + NS - 1) // NS

    @functools.partial(
        pl.kernel,
        out_type=jax.ShapeDtypeStruct((N, LANES), jnp.float32),
        mesh=_mesh(),
        compiler_params=_SC_PARAMS,
        scratch_types=[
            pltpu.VMEM_SHARED((N, LANES), jnp.float32),
            pltpu.VMEM((EK,), jnp.int32),
            pltpu.VMEM((EK, LANES), jnp.float32),
            pltpu.VMEM((RB, LANES), jnp.float32),
        ],
    )
    def deg_k(row_h, deg_h, acc, idx, ones, zbuf):
        c = lax.axis_index("c")
        s = lax.axis_index("s")
        one16 = jnp.full((LANES,), 1.0, jnp.float32)
        zero16 = jnp.zeros((LANES,), jnp.float32)

        @pl.loop(0, EK)
        def _(i):
            ones[i, :] = one16

        @pl.loop(0, RB)
        def _(i):
            zbuf[i, :] = zero16

        @pl.loop(0, trb)
        def _(t):
            rb = s + NS * t

            @pl.when(rb < nrb)
            def _():
                pltpu.sync_copy(zbuf, acc.at[pl.ds(rb * RB, RB)])

        plsc.subcore_barrier()

        @pl.loop(0, tch)
        def _(t):
            q = s + NS * t

            @pl.when(q < nch)
            def _():
                pltpu.sync_copy(row_h.at[pl.ds(q * EK, EK)], idx)
                pltpu.sync_copy(ones, acc.at[idx], add=True)

        plsc.subcore_barrier()

        @pl.when(c == 0)
        def _():
            @pl.loop(0, trb)
            def _(t):
                rb = s + NS * t

                @pl.when(rb < nrb)
                def _():
                    pltpu.sync_copy(acc.at[pl.ds(rb * RB, RB)], zbuf)
                    pltpu.sync_copy(zbuf, deg_h.at[pl.ds(rb * RB, RB)])
                    z16 = jnp.zeros((LANES,), jnp.float32)

                    @pl.loop(0, RB)
                    def _(i):
                        zbuf[i, :] = z16

    return deg_k


def _make_mlptab(N, DF, DH, C):
    RT = 1000
    nb = N // RT
    ch = C // NC
    f32 = jnp.float32

    def body(x_ref, w0_ref, b0_ref, w1_ref, b1_ref, deg_ref,
             z0_ref, a_ref, b2_ref, af_ref, bf2_ref):
        x = x_ref[...]
        h = lax.dot_general(x, w0_ref[...], (((1,), (1,)), ((), ())),
                            preferred_element_type=f32)
        h = jnp.maximum(h + b0_ref[...], 0.0)
        ini = lax.dot_general(h, w1_ref[...], (((1,), (1,)), ((), ())),
                              preferred_element_type=f32)
        ini = ini + b1_ref[pl.ds(pl.program_id(1), 1), :]
        dis = lax.rsqrt(jnp.maximum(deg_ref[...], 1e-24))
        z0 = dis * ini
        z0_ref[...] = z0
        a_ref[...] = ALPHA * z0
        af_ref[...] = ALPHA * ini
        b2_ref[...] = (1.0 - ALPHA) * dis * dis
        bf2_ref[...] = (1.0 - ALPHA) * dis

    return pl.pallas_call(
        body,
        grid=(nb, NC),
        in_specs=[
            pl.BlockSpec((RT, DF), lambda i, c: (i, 0)),
            pl.BlockSpec((DH, DF), lambda i, c: (0, 0)),
            pl.BlockSpec((1, DH), lambda i, c: (0, 0)),
            pl.BlockSpec((ch, DH), lambda i, c: (c, 0)),
            pl.BlockSpec((NC, ch), lambda i, c: (0, 0)),
            pl.BlockSpec((RT, LANES), lambda i, c: (i, 0)),
        ],
        out_specs=[
            pl.BlockSpec((RT, ch), lambda i, c: (c * nb + i, 0)),
            pl.BlockSpec((RT, ch), lambda i, c: (c * nb + i, 0)),
            pl.BlockSpec((RT, ch), lambda i, c: (i, 0)),
            pl.BlockSpec((RT, ch), lambda i, c: (c * nb + i, 0)),
            pl.BlockSpec((RT, ch), lambda i, c: (i, 0)),
        ],
        out_shape=[
            jax.ShapeDtypeStruct((NC * N, ch), f32),
            jax.ShapeDtypeStruct((NC * N, ch), f32),
            jax.ShapeDtypeStruct((N, ch), f32),
            jax.ShapeDtypeStruct((NC * N, ch), f32),
            jax.ShapeDtypeStruct((N, ch), f32),
        ],
    )


def _make_prop(N, E):
    nch = E // EK
    tch = (nch + NS - 1) // NS
    nrb = N // RB
    trb = (nrb + NS - 1) // NS
    ch = LANES
    shp2 = jax.ShapeDtypeStruct((NC * N, ch), jnp.float32)

    @functools.partial(
        pl.kernel,
        out_type=(shp2, shp2, shp2),
        mesh=_mesh(),
        compiler_params=_SC_PARAMS,
        scratch_types=[
            pltpu.VMEM_SHARED((N, ch), jnp.float32),
            pltpu.VMEM((EK,), jnp.int32),
            pltpu.VMEM((EK,), jnp.int32),
            pltpu.VMEM((EK, ch), jnp.float32),
            pltpu.VMEM((RB, ch), jnp.float32),
            pltpu.VMEM((RB, ch), jnp.float32),
            pltpu.VMEM((RB, ch), jnp.float32),
            pltpu.VMEM((RB, ch), jnp.float32),
        ],
    )
    def prop_k(z0, a_t, b2_t, af_t, bf2_t, colb_h, row_h,
               out_h, za_h, zb_h,
               acc, ecol, erow, gath, uacc, ua, ub, zb):
        c = lax.axis_index("c")
        s = lax.axis_index("s")
        coff = c * N
        cebase = c * E
        zero16 = jnp.zeros((ch,), jnp.float32)

        @pl.loop(0, RB)
        def _(i):
            zb[i, :] = zero16

        @pl.loop(0, trb)
        def _(t):
            rb = s + NS * t

            @pl.when(rb < nrb)
            def _():
                pltpu.sync_copy(zb, acc.at[pl.ds(rb * RB, RB)])

        plsc.subcore_barrier()

        srcs = [z0, za_h, zb_h, za_h, zb_h]
        dsts = [za_h, zb_h, za_h, zb_h, out_h]
        for p in range(N_PROP):
            zsrc, zdst = srcs[p], dsts[p]
            at_ = a_t if p < N_PROP - 1 else af_t
            bt_ = b2_t if p < N_PROP - 1 else bf2_t

            @pl.loop(0, tch)
            def _(t):
                q = s + NS * t

                @pl.when(q < nch)
                def _():
                    off = q * EK
                    pltpu.sync_copy(colb_h.at[pl.ds(cebase + off, EK)], ecol)
                    pltpu.sync_copy(zsrc.at[ecol], gath)
                    pltpu.sync_copy(row_h.at[pl.ds(off, EK)], erow)
                    pltpu.sync_copy(gath, acc.at[erow], add=True)

            plsc.subcore_barrier()

            @pl.loop(0, trb)
            def _(t):
                rb = s + NS * t

                @pl.when(rb < nrb)
                def _():
                    r0 = rb * RB
                    pltpu.sync_copy(acc.at[pl.ds(r0, RB)], uacc)
                    pltpu.sync_copy(at_.at[pl.ds(coff + r0, RB)], ua)
                    pltpu.sync_copy(bt_.at[pl.ds(r0, RB)], ub)

                    @pl.loop(0, RB)
                    def _(i):
                        ua[i, :] = ua[i, :] + ub[i, :] * uacc[i, :]

                    pltpu.sync_copy(ua, zdst.at[pl.ds(coff + r0, RB)])
                    pltpu.sync_copy(zb, acc.at[pl.ds(r0, RB)])

            plsc.subcore_barrier()

    return prop_k


def kernel(X, edge_index, W0, b0, W1, b1):
    N, DF = X.shape
    DH = W0.shape[0]
    C = W1.shape[0]
    E = edge_index.shape[1]
    row = edge_index[0]
    col = edge_index[1]
    colb = jnp.concatenate([col, col + N])
    b0r = b0.reshape(1, DH)
    b1r = b1.reshape(NC, C // NC)

    deg16 = _make_deg(N, E)(row)
    Z0, A, B2, AF, BF2 = _make_mlptab(N, DF, DH, C)(X, W0, b0r, W1, b1r, deg16)
    O, _, _ = _make_prop(N, E)(Z0, A, B2, AF, BF2, colb, row)
    return jnp.concatenate([O[:N], O[N:]], axis=1)
